# baseline (speedup 1.0000x reference)
import numpy as np

import jax
import jax.numpy as jnp
from jax.experimental import pallas as pl
from jax.experimental.pallas import tpu as pltpu


BATCH_TILE = 128

_STAGES = (
    (50, 3, 10, False),
    (24, 10, 10, True),
    (12, 10, 10, True),
    (6, 10, 10, False),
)


def _round128(v):
    return ((v + 127) // 128) * 128


def _stage_dims(win, cin, cout, pad1):
    wc = win - 2
    wp = wc // 2 + (1 if pad1 else 0)
    np_ = cout * wp
    off_b = _round128(np_)
    return wc, wp, np_, off_b


def _phase_selectors(win, wp, pad1):
    wc = win - 2
    if pad1:
        a = lambda p: max(2 * p - 1, 0)
        b = lambda p: min(2 * p, wc - 1)
    else:
        a = lambda p: 2 * p
        b = lambda p: 2 * p + 1
    ma = np.zeros((3, win, wp), np.float32)
    mb = np.zeros((3, win, wp), np.float32)
    for p in range(wp):
        for dx in range(3):
            ma[dx, a(p) + dx, p] = 1.0
            mb[dx, b(p) + dx, p] = 1.0
    return ma, mb


_KSLOT = 256


def _toeplitz_weights(w, win, cin, cout, pad1):
    _, wp, np_, off_b = _stage_dims(win, cin, cout, pad1)
    ma, mb = _phase_selectors(win, wp, pad1)
    k = cin * win
    out = jnp.zeros((3 * _KSLOT, off_b + np_), jnp.float32)
    for dy in range(3):
        wdy = w[dy]
        wa = jnp.einsum("dxp,dco->cxop", ma, wdy).reshape(k, np_)
        wb = jnp.einsum("dxp,dco->cxop", mb, wdy).reshape(k, np_)
        out = out.at[dy * _KSLOT:dy * _KSLOT + k, :np_].set(wa)
        out = out.at[dy * _KSLOT:dy * _KSLOT + k, off_b:].set(wb)
    return out


def _stage(src, tw_ref, b_ref, win, cin, cout, pad1, out_dtype=jnp.bfloat16):
    wc, wp, np_, off_b = _stage_dims(win, cin, cout, pad1)
    h = src.shape[0]
    bsz = src.shape[1]
    k = win * cin
    hc = h - 2

    zpad = jnp.zeros((hc * bsz, _KSLOT - k), jnp.bfloat16)
    parts = []
    for dy in range(3):
        parts += [src[dy:dy + hc].reshape(hc * bsz, k), zpad]
    aa = jnp.concatenate(parts, axis=1)
    acc = jnp.dot(aa, tw_ref[...], preferred_element_type=jnp.float32)
    cols = jnp.maximum(acc[:, :np_], acc[:, off_b:off_b + np_])
    cols = cols.reshape(hc, bsz, np_)
    if pad1:
        hp = hc // 2 + 1
        mid = cols[1:hc - 1].reshape(hp - 2, 2, bsz, np_)
        mid = jnp.maximum(mid[:, 0], mid[:, 1])
        pooled = jnp.concatenate([cols[0:1], mid, cols[hc - 1:hc]], axis=0)
    else:
        hp = hc // 2
        pairs = cols.reshape(hp, 2, bsz, np_)
        pooled = jnp.maximum(pairs[:, 0], pairs[:, 1])
    return jnp.maximum(pooled + b_ref[...], 0.0).astype(out_dtype)


def _net_kernel(x_ref,
                tw1_ref, b1_ref,
                tw2_ref, b2_ref,
                tw3_ref, b3_ref,
                tw4_ref, b4_ref,
                wf1_ref, bf1_ref,
                wf2_ref, bf2_ref,
                out_ref,
                a0_ref):
    for h in range(50):
        a0_ref[h] = jnp.concatenate(
            [x_ref[:, pl.ds(c * 2500 + h * 50, 50)] for c in range(3)],
            axis=1).astype(jnp.bfloat16)

    a1 = _stage(a0_ref, tw1_ref, b1_ref, *_STAGES[0])
    a2 = _stage(a1, tw2_ref, b2_ref, *_STAGES[1])
    a3 = _stage(a2, tw3_ref, b3_ref, *_STAGES[2])
    a4 = _stage(a3, tw4_ref, b4_ref, *_STAGES[3],
                out_dtype=jnp.float32)

    acc = jnp.dot(a4[0], wf1_ref[0], preferred_element_type=jnp.float32)
    acc = acc + jnp.dot(a4[1], wf1_ref[1],
                        preferred_element_type=jnp.float32)
    hidden = jnp.maximum(acc + bf1_ref[...], 0.0)
    logits = jnp.dot(hidden, wf2_ref[...],
                     preferred_element_type=jnp.float32) + bf2_ref[...]
    out_ref[...] = logits.astype(out_ref.dtype)


def kernel(conv1_w, conv1_b, conv2_w, conv2_b, conv3_w, conv3_b,
           conv4_w, conv4_b, fc1_w, fc1_b, fc2_w, fc2_b, x_nchw):
    n = x_nchw.shape[0]
    n_pad = ((n + BATCH_TILE - 1) // BATCH_TILE) * BATCH_TILE
    num_tiles = n_pad // BATCH_TILE

    x = x_nchw.astype(jnp.float32).reshape(n, 7500)
    if n_pad != n:
        x = jnp.pad(x, ((0, n_pad - n), (0, 0)))

    tws = [_toeplitz_weights(w, *s).astype(jnp.bfloat16)
           for w, s in zip((conv1_w, conv2_w, conv3_w, conv4_w), _STAGES)]

    def bias_row(b, win, cin, cout, pad1):
        wp = _stage_dims(win, cin, cout, pad1)[1]
        return jnp.repeat(b.astype(jnp.float32), wp).reshape(1, -1)

    biases = [bias_row(b, *s) for b, s in
              zip((conv1_b, conv2_b, conv3_b, conv4_b), _STAGES)]

    wf1 = jnp.transpose(fc1_w.reshape(10, 2, 2, 30),
                        (1, 0, 2, 3)).reshape(2, 20, 30)

    def const_spec(shape):
        return pl.BlockSpec(shape, lambda i: (0,) * len(shape))

    args = (x,
            tws[0], biases[0], tws[1], biases[1],
            tws[2], biases[2], tws[3], biases[3],
            wf1, fc1_b.reshape(1, 30).astype(jnp.float32),
            fc2_w, fc2_b.reshape(1, 10).astype(jnp.float32))

    in_specs = [pl.BlockSpec((BATCH_TILE, 7500), lambda i: (i, 0))]
    in_specs += [const_spec(a.shape) for a in args[1:]]

    flops_per_image = 2 * (48 * 48 * 27 * 10 + 22 * 22 * 90 * 10
                           + 10 * 10 * 90 * 10 + 4 * 4 * 90 * 10
                           + 40 * 30 + 30 * 10)
    bytes_accessed = sum(int(a.size) * 4 for a in args) + n_pad * 10 * 4

    out = pl.pallas_call(
        _net_kernel,
        out_shape=jax.ShapeDtypeStruct((n_pad, 10), jnp.float32),
        grid=(num_tiles,),
        in_specs=in_specs,
        out_specs=pl.BlockSpec((BATCH_TILE, 10), lambda i: (i, 0)),
        scratch_shapes=[pltpu.VMEM((50, BATCH_TILE, 150), jnp.bfloat16)],
        compiler_params=pltpu.CompilerParams(
            dimension_semantics=("parallel",),
            vmem_limit_bytes=100 * 1024 * 1024),
        cost_estimate=pl.CostEstimate(flops=n_pad * flops_per_image,
                                      transcendentals=0,
                                      bytes_accessed=int(bytes_accessed)),
    )(*args)
    return out[:n]


# speedup vs baseline: 1.2171x; 1.2171x over previous
import numpy as np

import jax
import jax.numpy as jnp
from jax.experimental import pallas as pl
from jax.experimental.pallas import tpu as pltpu


BATCH_TILE = 128

_STAGES = (
    (50, 3, 10, False),
    (24, 10, 10, True),
    (12, 10, 10, True),
    (6, 10, 10, False),
)


def _round128(v):
    return ((v + 127) // 128) * 128


def _stage_dims(win, cin, cout, pad1):
    wc = win - 2
    wp = wc // 2 + (1 if pad1 else 0)
    np_ = cout * wp
    off_b = _round128(np_)
    return wc, wp, np_, off_b


def _phase_selectors(win, wp, pad1):
    wc = win - 2
    if pad1:
        a = lambda p: max(2 * p - 1, 0)
        b = lambda p: min(2 * p, wc - 1)
    else:
        a = lambda p: 2 * p
        b = lambda p: 2 * p + 1
    ma = np.zeros((3, win, wp), np.float32)
    mb = np.zeros((3, win, wp), np.float32)
    for p in range(wp):
        for dx in range(3):
            ma[dx, a(p) + dx, p] = 1.0
            mb[dx, b(p) + dx, p] = 1.0
    return ma, mb


def _toeplitz_weights(w, win, cin, cout, pad1):
    _, wp, np_, off_b = _stage_dims(win, cin, cout, pad1)
    ma, mb = _phase_selectors(win, wp, pad1)
    out = []
    for dy in range(3):
        wdy = w[dy]
        wa = jnp.einsum("dxp,dco->cxop", ma, wdy).reshape(cin * win, np_)
        wb = jnp.einsum("dxp,dco->cxop", mb, wdy).reshape(cin * win, np_)
        pad = jnp.zeros((cin * win, off_b - np_), jnp.float32)
        out.append(jnp.concatenate([wa, pad, wb], axis=1))
    return jnp.stack(out)


def _stage(src, tw_ref, b_ref, win, cin, cout, pad1, out_dtype=jnp.bfloat16):
    wc, wp, np_, off_b = _stage_dims(win, cin, cout, pad1)
    h = src.shape[0]
    bsz = src.shape[1]
    k = win * cin
    hc = h - 2

    acc = None
    for dy in range(3):
        lhs = src[dy:dy + hc].reshape(hc * bsz, k)
        d = jnp.dot(lhs, tw_ref[dy], preferred_element_type=jnp.float32)
        acc = d if acc is None else acc + d
    cols = jnp.maximum(acc[:, :np_], acc[:, off_b:off_b + np_])
    cols = cols.reshape(hc, bsz, np_)
    if pad1:
        hp = hc // 2 + 1
        mid = cols[1:hc - 1].reshape(hp - 2, 2, bsz, np_)
        mid = jnp.maximum(mid[:, 0], mid[:, 1])
        pooled = jnp.concatenate([cols[0:1], mid, cols[hc - 1:hc]], axis=0)
    else:
        hp = hc // 2
        pairs = cols.reshape(hp, 2, bsz, np_)
        pooled = jnp.maximum(pairs[:, 0], pairs[:, 1])
    return jnp.maximum(pooled + b_ref[...], 0.0).astype(out_dtype)


def _net_kernel(x_ref,
                tw1_ref, b1_ref,
                tw2_ref, b2_ref,
                tw3_ref, b3_ref,
                tw4_ref, b4_ref,
                wf1_ref, bf1_ref,
                wf2_ref, bf2_ref,
                out_ref,
                a0_ref):
    for h in range(50):
        a0_ref[h] = jnp.concatenate(
            [x_ref[:, pl.ds(c * 2500 + h * 50, 50)] for c in range(3)],
            axis=1).astype(jnp.bfloat16)

    a1 = _stage(a0_ref, tw1_ref, b1_ref, *_STAGES[0])
    a2 = _stage(a1, tw2_ref, b2_ref, *_STAGES[1])
    a3 = _stage(a2, tw3_ref, b3_ref, *_STAGES[2])
    a4 = _stage(a3, tw4_ref, b4_ref, *_STAGES[3],
                out_dtype=jnp.float32)

    acc = jnp.dot(a4[0], wf1_ref[0], preferred_element_type=jnp.float32)
    acc = acc + jnp.dot(a4[1], wf1_ref[1],
                        preferred_element_type=jnp.float32)
    hidden = jnp.maximum(acc + bf1_ref[...], 0.0)
    logits = jnp.dot(hidden, wf2_ref[...],
                     preferred_element_type=jnp.float32) + bf2_ref[...]
    out_ref[...] = logits.astype(out_ref.dtype)


def kernel(conv1_w, conv1_b, conv2_w, conv2_b, conv3_w, conv3_b,
           conv4_w, conv4_b, fc1_w, fc1_b, fc2_w, fc2_b, x_nchw):
    n = x_nchw.shape[0]
    n_pad = ((n + BATCH_TILE - 1) // BATCH_TILE) * BATCH_TILE
    num_tiles = n_pad // BATCH_TILE

    x = x_nchw.astype(jnp.float32).reshape(n, 7500)
    if n_pad != n:
        x = jnp.pad(x, ((0, n_pad - n), (0, 0)))

    tws = [_toeplitz_weights(w, *s).astype(jnp.bfloat16)
           for w, s in zip((conv1_w, conv2_w, conv3_w, conv4_w), _STAGES)]

    def bias_row(b, win, cin, cout, pad1):
        wp = _stage_dims(win, cin, cout, pad1)[1]
        return jnp.repeat(b.astype(jnp.float32), wp).reshape(1, -1)

    biases = [bias_row(b, *s) for b, s in
              zip((conv1_b, conv2_b, conv3_b, conv4_b), _STAGES)]

    wf1 = jnp.transpose(fc1_w.reshape(10, 2, 2, 30),
                        (1, 0, 2, 3)).reshape(2, 20, 30)

    def const_spec(shape):
        return pl.BlockSpec(shape, lambda i: (0,) * len(shape))

    args = (x,
            tws[0], biases[0], tws[1], biases[1],
            tws[2], biases[2], tws[3], biases[3],
            wf1, fc1_b.reshape(1, 30).astype(jnp.float32),
            fc2_w, fc2_b.reshape(1, 10).astype(jnp.float32))

    in_specs = [pl.BlockSpec((BATCH_TILE, 7500), lambda i: (i, 0))]
    in_specs += [const_spec(a.shape) for a in args[1:]]

    flops_per_image = 2 * (48 * 48 * 27 * 10 + 22 * 22 * 90 * 10
                           + 10 * 10 * 90 * 10 + 4 * 4 * 90 * 10
                           + 40 * 30 + 30 * 10)
    bytes_accessed = sum(int(a.size) * 4 for a in args) + n_pad * 10 * 4

    out = pl.pallas_call(
        _net_kernel,
        out_shape=jax.ShapeDtypeStruct((n_pad, 10), jnp.float32),
        grid=(num_tiles,),
        in_specs=in_specs,
        out_specs=pl.BlockSpec((BATCH_TILE, 10), lambda i: (i, 0)),
        scratch_shapes=[pltpu.VMEM((50, BATCH_TILE, 150), jnp.bfloat16)],
        compiler_params=pltpu.CompilerParams(
            dimension_semantics=("parallel",),
            vmem_limit_bytes=100 * 1024 * 1024),
        cost_estimate=pl.CostEstimate(flops=n_pad * flops_per_image,
                                      transcendentals=0,
                                      bytes_accessed=int(bytes_accessed)),
    )(*args)
    return out[:n]
